# baseline (speedup 1.0000x reference)
import functools

import jax
import jax.numpy as jnp
from jax import lax
from jax.experimental import pallas as pl
from jax.experimental.pallas import tpu as pltpu
from jax.experimental.pallas import tpu_sc as plsc

B = 16384
D = 32
NC, NS = 2, 16
NW = NC * NS
PER_W = B // NW
PACK = 4
ROWS_W = PER_W // PACK

_mesh = plsc.VectorSubcoreMesh(core_axis_name="c", subcore_axis_name="s")


@functools.partial(
    pl.kernel,
    out_type=[
        jax.ShapeDtypeStruct((2, B // PACK, 128), jnp.float32),
        jax.ShapeDtypeStruct((4, B // PACK, 128), jnp.float32),
    ],
    mesh=_mesh,
    scratch_types=[
        pltpu.VMEM((PER_W,), jnp.int32),
        pltpu.VMEM((PER_W,), jnp.int32),
        pltpu.VMEM((PER_W,), jnp.int32),
        pltpu.VMEM((PER_W, 128), jnp.float32),
        pltpu.VMEM((ROWS_W, 128), jnp.float32),
        pltpu.SemaphoreType.DMA,
    ],
    compiler_params=pltpu.CompilerParams(use_tc_tiling_on_sc=False,
                                         needs_layout_passes=False),
)
def _sc_gather(t_tab, a_tab, t_idx, a_idx, t_out, a_out,
               idx_v, hi_v, m_v, big_v, sel_v, sem):
    wid = lax.axis_index("s") * NC + lax.axis_index("c")
    base = wid * PER_W
    obase = wid * ROWS_W
    lanes16 = lax.iota(jnp.int32, 16)

    def one_slot(tab, idx_hbm, out_hbm, j):
        pltpu.sync_copy(idx_hbm.at[j].at[pl.ds(base, PER_W)], idx_v)

        @pl.loop(0, PER_W // 16)
        def _(i):
            chunk = idx_v[pl.ds(i * 16, 16)]
            hi_v[pl.ds(i * 16, 16)] = chunk >> 2
            m_v[pl.ds(i * 16, 16)] = chunk & 3

        pltpu.async_copy(tab.at[hi_v], big_v, sem).wait()

        @pl.loop(0, PER_W)
        def _(b):
            r = b >> 2
            k = b & 3
            bvec = jnp.full((16,), b, jnp.int32)
            m = plsc.load_gather(m_v, [bvec])
            col0 = (m << 5) + lanes16
            lo = plsc.load_gather(big_v, [bvec, col0])
            hi = plsc.load_gather(big_v, [bvec, col0 + 16])
            sel_v[r, pl.ds(k * 32, 16)] = lo
            sel_v[r, pl.ds(k * 32 + 16, 16)] = hi

        pltpu.sync_copy(sel_v, out_hbm.at[j].at[pl.ds(obase, ROWS_W)])

    for j in range(2):
        one_slot(t_tab, t_idx, t_out, j)
    for j in range(4):
        one_slot(a_tab, a_idx, a_out, j)


NBP = 1024


def _combine_body(t_ref, a_ref, wd_ref, b_ref, o_ref):
    acc = jnp.dot(t_ref[0], wd_ref[0], preferred_element_type=jnp.float32)
    acc = acc + jnp.dot(t_ref[1], wd_ref[1], preferred_element_type=jnp.float32)
    for j in range(4):
        acc = acc + jnp.dot(a_ref[j], wd_ref[2 + j],
                            preferred_element_type=jnp.float32)
    o_ref[...] = acc + b_ref[...]


def _combine(t_emb, a_emb, wd, bp):
    return pl.pallas_call(
        _combine_body,
        grid=(B // PACK // NBP,),
        in_specs=[
            pl.BlockSpec((2, NBP, 128), lambda i: (0, i, 0)),
            pl.BlockSpec((4, NBP, 128), lambda i: (0, i, 0)),
            pl.BlockSpec((6, 128, 128), lambda i: (0, 0, 0)),
            pl.BlockSpec((1, 128), lambda i: (0, 0)),
        ],
        out_specs=pl.BlockSpec((NBP, 128), lambda i: (i, 0)),
        out_shape=jax.ShapeDtypeStruct((B // PACK, 128), jnp.float32),
    )(t_emb, a_emb, wd, bp)


def kernel(type_ids, ability_ids, type_table, ability_table, W, b):
    t_idx = type_ids.T.astype(jnp.int32)
    a_idx = ability_ids.T.astype(jnp.int32)
    t_tab = type_table.reshape(1000 // PACK, 128)
    a_tab = ability_table.reshape(1000000 // PACK, 128)
    t_emb, a_emb = _sc_gather(t_tab, a_tab, t_idx, a_idx)

    wt = W.T
    eye4 = jnp.eye(PACK, dtype=W.dtype)
    wd = jnp.stack([jnp.kron(eye4, wt[j * D:(j + 1) * D, :])
                    for j in range(6)])
    bp = jnp.tile(b, PACK).reshape(1, 128)
    out = _combine(t_emb, a_emb, wd, bp)
    return out.reshape(B, D)
